# baseline (speedup 1.0000x reference)
import functools
import math

import jax
import jax.numpy as jnp
from jax import lax
from jax.experimental import pallas as pl
from jax.experimental.pallas import tpu as pltpu
from jax.experimental.pallas import tpu_sc as plsc

_VOCAB = 100000
_MAXLEN = 2048
_HID = 128
_BATCH = 64

_INFO = plsc.get_sparse_core_info()
_NC = _INFO.num_cores
_NS = _INFO.num_subcores
_NW = _NC * _NS
_TPW = _MAXLEN // _NW
_LANES = _INFO.num_lanes
_SCALE = math.sqrt(float(_HID))
_K = 4
_NCHUNK = _BATCH // _K
_NSLOT = 3
_VPR = _TPW // _LANES


def _body(x_hbm, tok_hbm, pos_hbm, out_hbm, idx_v, pidx, gbuf, pos_v,
          gsems, ssems):
    wid = lax.axis_index("s") * _NC + lax.axis_index("c")
    t0 = wid * _TPW
    c0 = (wid // 2) * (2 * _TPW)
    off = (wid % 2) * _TPW

    pltpu.sync_copy(pos_hbm.at[pl.ds(t0, _TPW)], pos_v)
    pltpu.sync_copy(x_hbm.at[:, pl.ds(c0, 2 * _TPW)], idx_v)

    def repack(p, carry):
        for h in range(2):
            for j in range(_VPR):
                pidx[p, pl.ds(h * _TPW + j * _LANES, _LANES)] = (
                    idx_v[2 * p + h, pl.ds(off + j * _LANES, _LANES)])
        return carry

    lax.fori_loop(0, _BATCH // 2, repack, 0)

    def issue_gathers(c):
        s = c % _NSLOT
        for h in range(_K // 2):
            pltpu.async_copy(tok_hbm.at[pidx.at[2 * c + h]],
                             gbuf.at[s, pl.ds(h * 2 * _TPW, 2 * _TPW)],
                             gsems[s])

    def wait_gathers(c):
        s = c % _NSLOT
        for h in range(_K // 2):
            pltpu.make_async_copy(tok_hbm.at[pidx.at[2 * c + h]],
                                  gbuf.at[s, pl.ds(h * 2 * _TPW, 2 * _TPW)],
                                  gsems[s]).wait()

    def store_copies(c):
        s = c % _NSLOT
        return [pltpu.make_async_copy(
                    gbuf.at[s, pl.ds(k * _TPW, _TPW)],
                    out_hbm.at[c * _K + k, pl.ds(t0, _TPW)],
                    ssems[s])
                for k in range(_K)]

    def fma(c):
        s = c % _NSLOT

        def fma_row(r, carry):
            pv = [pos_v[r, pl.ds(j * _LANES, _LANES)]
                  for j in range(_HID // _LANES)]
            for k in range(_K):
                for j in range(_HID // _LANES):
                    sl = pl.ds(j * _LANES, _LANES)
                    gbuf[s, k * _TPW + r, sl] = (
                        gbuf[s, k * _TPW + r, sl] * _SCALE + pv[j])
            return carry

        lax.fori_loop(0, _TPW, fma_row, 0)

    issue_gathers(0)
    issue_gathers(1)
    for c in range(_NCHUNK):
        wait_gathers(c)
        fma(c)
        for cp in store_copies(c):
            cp.start()
        if c + 2 < _NCHUNK:
            if c >= 1:
                for cp in store_copies(c - 1):
                    cp.wait()
            issue_gathers(c + 2)
    for c in range(_NCHUNK - _NSLOT, _NCHUNK):
        for cp in store_copies(c):
            cp.wait()


@jax.jit
def kernel(x, token_table, pos_table):
    x = x.astype(jnp.int32)
    mesh = plsc.VectorSubcoreMesh(core_axis_name="c", subcore_axis_name="s")
    f = functools.partial(
        pl.kernel,
        mesh=mesh,
        out_type=jax.ShapeDtypeStruct((_BATCH, _MAXLEN, _HID), jnp.float32),
        scratch_types=[
            pltpu.VMEM((_BATCH, 2 * _TPW), jnp.int32),
            pltpu.VMEM((_BATCH // 2, 2 * _TPW), jnp.int32),
            pltpu.VMEM((_NSLOT, _K * _TPW, _HID), jnp.float32),
            pltpu.VMEM((_TPW, _HID), jnp.float32),
            [pltpu.SemaphoreType.DMA] * _NSLOT,
            [pltpu.SemaphoreType.DMA] * _NSLOT,
        ],
    )(_body)
    return f(x, token_table, pos_table)


# speedup vs baseline: 1.0707x; 1.0707x over previous
import functools
import math

import jax
import jax.numpy as jnp
from jax import lax
from jax.experimental import pallas as pl
from jax.experimental.pallas import tpu as pltpu
from jax.experimental.pallas import tpu_sc as plsc

_VOCAB = 100000
_MAXLEN = 2048
_HID = 128
_BATCH = 64

_INFO = plsc.get_sparse_core_info()
_NC = _INFO.num_cores
_NS = _INFO.num_subcores
_NW = _NC * _NS
_TPW = _MAXLEN // _NW
_LANES = _INFO.num_lanes
_SCALE = math.sqrt(float(_HID))
_NBUF = 6


def _body(x_hbm, tok_hbm, pos_hbm, out_hbm, idx_v, gbuf, obuf, pos_v,
          gsems, ssems):
    wid = lax.axis_index("s") * _NC + lax.axis_index("c")
    t0 = wid * _TPW
    c0 = (wid // 2) * (2 * _TPW)
    off = (wid % 2) * _TPW

    pltpu.sync_copy(pos_hbm.at[pl.ds(t0, _TPW)], pos_v)
    pltpu.sync_copy(x_hbm.at[:, pl.ds(c0, 2 * _TPW)], idx_v)

    for s in range(_NBUF):
        pltpu.async_copy(tok_hbm.at[idx_v.at[s, pl.ds(off, _TPW)]],
                         gbuf.at[s], gsems[s])

    def group(i, carry):
        for s in range(_NBUF):
            b = i * _NBUF + s
            pltpu.make_async_copy(tok_hbm.at[idx_v.at[s, pl.ds(off, _TPW)]],
                                  gbuf.at[s], gsems[s]).wait()
            @pl.when(i > 0)
            def _wait_store():
                pltpu.make_async_copy(obuf.at[s],
                                      out_hbm.at[b - _NBUF, pl.ds(t0, _TPW)],
                                      ssems[s]).wait()

            def fma_row(r, c):
                for j in range(_HID // _LANES):
                    sl = pl.ds(j * _LANES, _LANES)
                    obuf[s, r, sl] = gbuf[s, r, sl] * _SCALE + pos_v[r, sl]
                return c

            lax.fori_loop(0, _TPW, fma_row, 0)
            pltpu.async_copy(obuf.at[s], out_hbm.at[b, pl.ds(t0, _TPW)],
                             ssems[s])
            @pl.when(i < _BATCH // _NBUF)
            def _next_gather():
                @pl.when(b + _NBUF < _BATCH)
                def _():
                    pltpu.async_copy(
                        tok_hbm.at[idx_v.at[b + _NBUF, pl.ds(off, _TPW)]],
                        gbuf.at[s], gsems[s])
        return carry

    lax.fori_loop(0, _BATCH // _NBUF, group, 0)

    tail_start = (_BATCH // _NBUF) * _NBUF
    for b in range(tail_start, _BATCH):
        s = b % _NBUF
        pltpu.make_async_copy(tok_hbm.at[idx_v.at[s, pl.ds(off, _TPW)]],
                              gbuf.at[s], gsems[s]).wait()
        pltpu.make_async_copy(obuf.at[s],
                              out_hbm.at[b - _NBUF, pl.ds(t0, _TPW)],
                              ssems[s]).wait()

        def fma_row_t(r, c):
            for j in range(_HID // _LANES):
                sl = pl.ds(j * _LANES, _LANES)
                obuf[s, r, sl] = gbuf[s, r, sl] * _SCALE + pos_v[r, sl]
            return c

        lax.fori_loop(0, _TPW, fma_row_t, 0)
        pltpu.async_copy(obuf.at[s], out_hbm.at[b, pl.ds(t0, _TPW)],
                         ssems[s])
    for b in range(_BATCH - _NBUF, _BATCH):
        s = b % _NBUF
        pltpu.make_async_copy(obuf.at[s], out_hbm.at[b, pl.ds(t0, _TPW)],
                              ssems[s]).wait()


@jax.jit
def kernel(x, token_table, pos_table):
    x = x.astype(jnp.int32)
    mesh = plsc.VectorSubcoreMesh(core_axis_name="c", subcore_axis_name="s")
    f = functools.partial(
        pl.kernel,
        mesh=mesh,
        out_type=jax.ShapeDtypeStruct((_BATCH, _MAXLEN, _HID), jnp.float32),
        scratch_types=[
            pltpu.VMEM((_BATCH, 2 * _TPW), jnp.int32),
            pltpu.VMEM((_NBUF, _TPW, _HID), jnp.float32),
            pltpu.VMEM((_NBUF, _TPW, _HID), jnp.float32),
            pltpu.VMEM((_TPW, _HID), jnp.float32),
            [pltpu.SemaphoreType.DMA] * _NBUF,
            [pltpu.SemaphoreType.DMA] * _NBUF,
        ],
    )(_body)
    return f(x, token_table, pos_table)


# speedup vs baseline: 1.0773x; 1.0062x over previous
import functools
import math

import jax
import jax.numpy as jnp
from jax import lax
from jax.experimental import pallas as pl
from jax.experimental.pallas import tpu as pltpu
from jax.experimental.pallas import tpu_sc as plsc

_VOCAB = 100000
_MAXLEN = 2048
_HID = 128
_BATCH = 64

_INFO = plsc.get_sparse_core_info()
_NC = _INFO.num_cores
_NS = _INFO.num_subcores
_NW = _NC * _NS
_TPW = _MAXLEN // _NW
_LANES = _INFO.num_lanes
_SCALE = math.sqrt(float(_HID))
_NBUF = 6


def _body(x_hbm, tok_hbm, pos_hbm, out_hbm, idx_v, gbuf, obuf, pos_v,
          gsems, ssems):
    wid = lax.axis_index("s") * _NC + lax.axis_index("c")
    t0 = wid * _TPW
    c0 = (wid // 2) * (2 * _TPW)
    off = (wid % 2) * _TPW

    pltpu.sync_copy(pos_hbm.at[pl.ds(t0, _TPW)], pos_v)
    pltpu.sync_copy(x_hbm.at[:, pl.ds(c0, 2 * _TPW)], idx_v)

    for s in range(_NBUF):
        pltpu.async_copy(tok_hbm.at[idx_v.at[s, pl.ds(off, _TPW)]],
                         gbuf.at[s], gsems[s])

    def group(i, carry):
        for s in range(0, _NBUF, 2):
            bs = [i * _NBUF + s, i * _NBUF + s + 1]
            for u, b in zip((s, s + 1), bs):
                pltpu.make_async_copy(
                    tok_hbm.at[idx_v.at[u, pl.ds(off, _TPW)]],
                    gbuf.at[u], gsems[u]).wait()

                @pl.when(i > 0)
                def _wait_store():
                    pltpu.make_async_copy(
                        obuf.at[u], out_hbm.at[b - _NBUF, pl.ds(t0, _TPW)],
                        ssems[u]).wait()

            def fma_row(r, c):
                pv = [pos_v[r, pl.ds(j * _LANES, _LANES)]
                      for j in range(_HID // _LANES)]
                for u in (s, s + 1):
                    for j in range(_HID // _LANES):
                        sl = pl.ds(j * _LANES, _LANES)
                        obuf[u, r, sl] = gbuf[u, r, sl] * _SCALE + pv[j]
                return c

            lax.fori_loop(0, _TPW, fma_row, 0)
            for u, b in zip((s, s + 1), bs):
                pltpu.async_copy(obuf.at[u], out_hbm.at[b, pl.ds(t0, _TPW)],
                                 ssems[u])
                @pl.when(b + _NBUF < _BATCH)
                def _next_gather():
                    pltpu.async_copy(
                        tok_hbm.at[idx_v.at[b + _NBUF, pl.ds(off, _TPW)]],
                        gbuf.at[u], gsems[u])
        return carry

    lax.fori_loop(0, _BATCH // _NBUF, group, 0)

    tail_start = (_BATCH // _NBUF) * _NBUF
    for b in range(tail_start, _BATCH):
        s = b % _NBUF
        pltpu.make_async_copy(tok_hbm.at[idx_v.at[s, pl.ds(off, _TPW)]],
                              gbuf.at[s], gsems[s]).wait()
        pltpu.make_async_copy(obuf.at[s],
                              out_hbm.at[b - _NBUF, pl.ds(t0, _TPW)],
                              ssems[s]).wait()

        def fma_row_t(r, c):
            for j in range(_HID // _LANES):
                sl = pl.ds(j * _LANES, _LANES)
                obuf[s, r, sl] = gbuf[s, r, sl] * _SCALE + pos_v[r, sl]
            return c

        lax.fori_loop(0, _TPW, fma_row_t, 0)
        pltpu.async_copy(obuf.at[s], out_hbm.at[b, pl.ds(t0, _TPW)],
                         ssems[s])
    for b in range(_BATCH - _NBUF, _BATCH):
        s = b % _NBUF
        pltpu.make_async_copy(obuf.at[s], out_hbm.at[b, pl.ds(t0, _TPW)],
                              ssems[s]).wait()


@jax.jit
def kernel(x, token_table, pos_table):
    x = x.astype(jnp.int32)
    mesh = plsc.VectorSubcoreMesh(core_axis_name="c", subcore_axis_name="s")
    f = functools.partial(
        pl.kernel,
        mesh=mesh,
        out_type=jax.ShapeDtypeStruct((_BATCH, _MAXLEN, _HID), jnp.float32),
        scratch_types=[
            pltpu.VMEM((_BATCH, 2 * _TPW), jnp.int32),
            pltpu.VMEM((_NBUF, _TPW, _HID), jnp.float32),
            pltpu.VMEM((_NBUF, _TPW, _HID), jnp.float32),
            pltpu.VMEM((_TPW, _HID), jnp.float32),
            [pltpu.SemaphoreType.DMA] * _NBUF,
            [pltpu.SemaphoreType.DMA] * _NBUF,
        ],
    )(_body)
    return f(x, token_table, pos_table)


# speedup vs baseline: 1.0790x; 1.0016x over previous
import functools
import math

import jax
import jax.numpy as jnp
from jax import lax
from jax.experimental import pallas as pl
from jax.experimental.pallas import tpu as pltpu
from jax.experimental.pallas import tpu_sc as plsc

_VOCAB = 100000
_MAXLEN = 2048
_HID = 128
_BATCH = 64

_INFO = plsc.get_sparse_core_info()
_NC = _INFO.num_cores
_NS = _INFO.num_subcores
_NW = _NC * _NS
_TPW = _MAXLEN // _NW
_LANES = _INFO.num_lanes
_SCALE = math.sqrt(float(_HID))
_NBUF = 6


def _body(x_hbm, tok_hbm, pos_hbm, out_hbm, idx_v, gbuf, obuf, pos_v,
          gsems, ssems):
    wid = lax.axis_index("s") * _NC + lax.axis_index("c")
    t0 = wid * _TPW
    c0 = (wid // 2) * (2 * _TPW)
    off = (wid % 2) * _TPW

    pltpu.sync_copy(x_hbm.at[:, pl.ds(c0, 2 * _TPW)], idx_v)

    for s in range(_NBUF):
        pltpu.async_copy(tok_hbm.at[idx_v.at[s, pl.ds(off, _TPW)]],
                         gbuf.at[s], gsems[s])

    pltpu.sync_copy(pos_hbm.at[pl.ds(t0, _TPW)], pos_v)

    def group(i, carry):
        for s in range(0, _NBUF, 2):
            bs = [i * _NBUF + s, i * _NBUF + s + 1]
            for u, b in zip((s, s + 1), bs):
                pltpu.make_async_copy(
                    tok_hbm.at[idx_v.at[u, pl.ds(off, _TPW)]],
                    gbuf.at[u], gsems[u]).wait()

                @pl.when(i > 0)
                def _wait_store():
                    pltpu.make_async_copy(
                        obuf.at[u], out_hbm.at[b - _NBUF, pl.ds(t0, _TPW)],
                        ssems[u]).wait()

            def fma_row(r, c):
                pv = [pos_v[r, pl.ds(j * _LANES, _LANES)]
                      for j in range(_HID // _LANES)]
                for u in (s, s + 1):
                    for j in range(_HID // _LANES):
                        sl = pl.ds(j * _LANES, _LANES)
                        obuf[u, r, sl] = gbuf[u, r, sl] * _SCALE + pv[j]
                return c

            lax.fori_loop(0, _TPW, fma_row, 0)
            for u, b in zip((s, s + 1), bs):
                pltpu.async_copy(obuf.at[u], out_hbm.at[b, pl.ds(t0, _TPW)],
                                 ssems[u])
                @pl.when(b + _NBUF < _BATCH)
                def _next_gather():
                    pltpu.async_copy(
                        tok_hbm.at[idx_v.at[b + _NBUF, pl.ds(off, _TPW)]],
                        gbuf.at[u], gsems[u])
        return carry

    lax.fori_loop(0, _BATCH // _NBUF, group, 0)

    tail_start = (_BATCH // _NBUF) * _NBUF
    for b in range(tail_start, _BATCH):
        s = b % _NBUF
        pltpu.make_async_copy(tok_hbm.at[idx_v.at[s, pl.ds(off, _TPW)]],
                              gbuf.at[s], gsems[s]).wait()
        pltpu.make_async_copy(obuf.at[s],
                              out_hbm.at[b - _NBUF, pl.ds(t0, _TPW)],
                              ssems[s]).wait()

        def fma_row_t(r, c):
            for j in range(_HID // _LANES):
                sl = pl.ds(j * _LANES, _LANES)
                obuf[s, r, sl] = gbuf[s, r, sl] * _SCALE + pos_v[r, sl]
            return c

        lax.fori_loop(0, _TPW, fma_row_t, 0)
        pltpu.async_copy(obuf.at[s], out_hbm.at[b, pl.ds(t0, _TPW)],
                         ssems[s])
    for b in range(_BATCH - _NBUF, _BATCH):
        s = b % _NBUF
        pltpu.make_async_copy(obuf.at[s], out_hbm.at[b, pl.ds(t0, _TPW)],
                              ssems[s]).wait()


@jax.jit
def kernel(x, token_table, pos_table):
    x = x.astype(jnp.int32)
    mesh = plsc.VectorSubcoreMesh(core_axis_name="c", subcore_axis_name="s")
    f = functools.partial(
        pl.kernel,
        mesh=mesh,
        out_type=jax.ShapeDtypeStruct((_BATCH, _MAXLEN, _HID), jnp.float32),
        scratch_types=[
            pltpu.VMEM((_BATCH, 2 * _TPW), jnp.int32),
            pltpu.VMEM((_NBUF, _TPW, _HID), jnp.float32),
            pltpu.VMEM((_NBUF, _TPW, _HID), jnp.float32),
            pltpu.VMEM((_TPW, _HID), jnp.float32),
            [pltpu.SemaphoreType.DMA] * _NBUF,
            [pltpu.SemaphoreType.DMA] * _NBUF,
        ],
    )(_body)
    return f(x, token_table, pos_table)
